# baseline (speedup 1.0000x reference)
import dataclasses
import functools

import jax
import jax.numpy as jnp
from jax import lax
from jax.experimental import pallas as pl
from jax.experimental.pallas import tpu as pltpu
from jax.experimental.pallas import tpu_sc as plsc

_L = 16
_NC = 2
_NS = 16


def _gather_rows(x2d, idx2d):
    rows, k = x2d.shape
    _, n = idx2d.shape
    nw = _NC * _NS
    rows_per_w = rows // nw
    mesh = plsc.VectorSubcoreMesh(core_axis_name="c", subcore_axis_name="s")

    cp = pltpu.CompilerParams()
    if "needs_layout_passes" in pltpu.CompilerParams.__dataclass_fields__:
        cp = dataclasses.replace(cp, needs_layout_passes=False)

    @functools.partial(
        pl.kernel,
        compiler_params=cp,
        out_type=jax.ShapeDtypeStruct((rows, n), jnp.float32),
        mesh=mesh,
        scratch_types=[
            pltpu.VMEM((k,), jnp.float32),
            pltpu.VMEM((n,), jnp.int32),
            pltpu.VMEM((n,), jnp.float32),
        ],
    )
    def sc_kernel(x_hbm, idx_hbm, out_hbm, xrow, irow, orow):
        wid = lax.axis_index("s") * _NC + lax.axis_index("c")
        base = wid * rows_per_w

        @pl.loop(0, rows_per_w)
        def _(j):
            r = base + j
            pltpu.sync_copy(x_hbm.at[r], xrow)
            pltpu.sync_copy(idx_hbm.at[r], irow)

            @pl.loop(0, n, step=_L)
            def _(i):
                iv = irow[pl.ds(i, _L)]
                orow[pl.ds(i, _L)] = plsc.load_gather(xrow, [iv])

            pltpu.sync_copy(orow, out_hbm.at[r])

    return sc_kernel(x2d, idx2d)


def kernel(x, indices):
    b, h, k = x.shape
    n = indices.shape[-1]
    x2d = x.reshape(b * h, k)
    idx2d = indices.astype(jnp.int32).reshape(b * h, n)
    out = _gather_rows(x2d, idx2d)
    return out.reshape(b, h, n)


# speedup vs baseline: 2.1991x; 2.1991x over previous
import dataclasses
import functools

import jax
import jax.numpy as jnp
from jax import lax
from jax.experimental import pallas as pl
from jax.experimental.pallas import tpu as pltpu
from jax.experimental.pallas import tpu_sc as plsc

_L = 16
_NC = 2
_NS = 16


def _gather_rows(x2d, idx2d):
    rows, k = x2d.shape
    _, n = idx2d.shape
    nw = _NC * _NS
    rows_per_w = rows // nw
    mesh = plsc.VectorSubcoreMesh(core_axis_name="c", subcore_axis_name="s")

    cp = pltpu.CompilerParams()
    if "needs_layout_passes" in pltpu.CompilerParams.__dataclass_fields__:
        cp = dataclasses.replace(cp, needs_layout_passes=False)

    @functools.partial(
        pl.kernel,
        compiler_params=cp,
        out_type=jax.ShapeDtypeStruct((rows, n), jnp.float32),
        mesh=mesh,
        scratch_types=[
            pltpu.VMEM((k,), jnp.float32),
            pltpu.VMEM((k,), jnp.float32),
            pltpu.VMEM((n,), jnp.int32),
            pltpu.VMEM((n,), jnp.int32),
            pltpu.VMEM((n,), jnp.float32),
            pltpu.VMEM((n,), jnp.float32),
            pltpu.SemaphoreType.DMA,
            pltpu.SemaphoreType.DMA,
        ],
    )
    def sc_kernel(x_hbm, idx_hbm, out_hbm, xb0, xb1, ib0, ib1, ob0, ob1,
                  sem0, sem1):
        wid = lax.axis_index("s") * _NC + lax.axis_index("c")
        base = wid * rows_per_w
        xbufs, ibufs, obufs, sems = (xb0, xb1), (ib0, ib1), (ob0, ob1), (sem0, sem1)

        def start_in(r, b):
            pltpu.make_async_copy(x_hbm.at[base + r], xbufs[b], sems[b]).start()
            pltpu.make_async_copy(idx_hbm.at[base + r], ibufs[b], sems[b]).start()

        def wait_in(b):
            pltpu.make_async_copy(x_hbm.at[base], xbufs[b], sems[b]).wait()
            pltpu.make_async_copy(idx_hbm.at[base], ibufs[b], sems[b]).wait()

        start_in(0, 0)

        @pl.loop(0, rows_per_w, step=2)
        def _(j):
            for b in range(2):
                r = j + b

                @pl.when(r + 1 < rows_per_w)
                def _():
                    start_in(r + 1, 1 - b)

                wait_in(b)

                @pl.loop(0, n, step=4 * _L)
                def _(i):
                    for u in range(4):
                        o = i + u * _L
                        iv = ibufs[b][pl.ds(o, _L)]
                        obufs[b][pl.ds(o, _L)] = plsc.load_gather(
                            xbufs[b], [iv])

                pltpu.sync_copy(obufs[b], out_hbm.at[base + r])

    return sc_kernel(x2d, idx2d)


def kernel(x, indices):
    b, h, k = x.shape
    n = indices.shape[-1]
    x2d = x.reshape(b * h, k)
    idx2d = indices.astype(jnp.int32).reshape(b * h, n)
    out = _gather_rows(x2d, idx2d)
    return out.reshape(b, h, n)


# speedup vs baseline: 2.2589x; 1.0272x over previous
import dataclasses
import functools

import jax
import jax.numpy as jnp
from jax import lax
from jax.experimental import pallas as pl
from jax.experimental.pallas import tpu as pltpu
from jax.experimental.pallas import tpu_sc as plsc

_L = 16
_NC = 2
_NS = 16


def _gather_rows(x2d, idx2d):
    rows, k = x2d.shape
    _, n = idx2d.shape
    nw = _NC * _NS
    rows_per_w = rows // nw
    mesh = plsc.VectorSubcoreMesh(core_axis_name="c", subcore_axis_name="s")

    cp = pltpu.CompilerParams()
    if "needs_layout_passes" in pltpu.CompilerParams.__dataclass_fields__:
        cp = dataclasses.replace(cp, needs_layout_passes=False)

    @functools.partial(
        pl.kernel,
        compiler_params=cp,
        out_type=jax.ShapeDtypeStruct((rows, n), jnp.float32),
        mesh=mesh,
        scratch_types=[
            pltpu.VMEM((k,), jnp.float32),
            pltpu.VMEM((k,), jnp.float32),
            pltpu.VMEM((n,), jnp.int32),
            pltpu.VMEM((n,), jnp.int32),
            pltpu.VMEM((n,), jnp.float32),
            pltpu.VMEM((n,), jnp.float32),
            pltpu.SemaphoreType.DMA,
            pltpu.SemaphoreType.DMA,
            pltpu.SemaphoreType.DMA,
            pltpu.SemaphoreType.DMA,
        ],
    )
    def sc_kernel(x_hbm, idx_hbm, out_hbm, xb0, xb1, ib0, ib1, ob0, ob1,
                  sem0, sem1, osem0, osem1):
        wid = lax.axis_index("s") * _NC + lax.axis_index("c")
        base = wid * rows_per_w
        xbufs, ibufs, obufs, sems = (xb0, xb1), (ib0, ib1), (ob0, ob1), (sem0, sem1)
        osems = (osem0, osem1)

        def start_in(r, b):
            pltpu.make_async_copy(x_hbm.at[base + r], xbufs[b], sems[b]).start()
            pltpu.make_async_copy(idx_hbm.at[base + r], ibufs[b], sems[b]).start()

        def wait_in(b):
            pltpu.make_async_copy(x_hbm.at[base], xbufs[b], sems[b]).wait()
            pltpu.make_async_copy(idx_hbm.at[base], ibufs[b], sems[b]).wait()

        def start_out(r, b):
            pltpu.make_async_copy(obufs[b], out_hbm.at[base + r], osems[b]).start()

        def wait_out(b):
            pltpu.make_async_copy(obufs[b], out_hbm.at[base], osems[b]).wait()

        start_in(0, 0)

        @pl.loop(0, rows_per_w, step=2)
        def _(j):
            for b in range(2):
                r = j + b

                @pl.when(r + 1 < rows_per_w)
                def _():
                    start_in(r + 1, 1 - b)

                wait_in(b)

                @pl.when(r >= 2)
                def _():
                    wait_out(b)

                @pl.loop(0, n, step=8 * _L)
                def _(i):
                    for u in range(8):
                        o = i + u * _L
                        iv = ibufs[b][pl.ds(o, _L)]
                        obufs[b][pl.ds(o, _L)] = plsc.load_gather(
                            xbufs[b], [iv])

                start_out(r, b)

        wait_out(0)
        wait_out(1)

    return sc_kernel(x2d, idx2d)


def kernel(x, indices):
    b, h, k = x.shape
    n = indices.shape[-1]
    x2d = x.reshape(b * h, k)
    idx2d = indices.astype(jnp.int32).reshape(b * h, n)
    out = _gather_rows(x2d, idx2d)
    return out.reshape(b, h, n)
